# Initial kernel scaffold; baseline (speedup 1.0000x reference)
import jax
import jax.numpy as jnp
from jax.experimental import pallas as pl


def kernel(node_ids, attrs, struct_table, attr_table, attr_fc_w, attr_fc_b, fusion_w, fusion_b):
    raise NotImplementedError("write your pallas kernel here")


# speedup vs baseline: 2.4091x; 2.4091x over previous
import functools

import jax
import jax.numpy as jnp
from jax import lax
from jax.experimental import pallas as pl
from jax.experimental.pallas import tpu as pltpu
from jax.experimental.pallas import tpu_sc as plsc

B = 16384
D = 128
NC = 2
NS = 16
NW = NC * NS
BPW = B // NW
CH = 128
NCH = BPW // CH


def _lut_body(at_ref, wa_ref, ba_ref, w2_ref, bf_ref, out_ref):
    feat = jnp.dot(at_ref[...], wa_ref[...], preferred_element_type=jnp.float32)
    feat = feat + ba_ref[...]
    out_ref[...] = jnp.dot(feat, w2_ref[...], preferred_element_type=jnp.float32) + bf_ref[...]


def _fuse_body(semb_ref, w1_ref, acont_ref, out_ref):
    out_ref[...] = (
        jnp.dot(semb_ref[...], w1_ref[...], preferred_element_type=jnp.float32)
        + acont_ref[...]
    )


def _sc_gather(node_hbm, attr_hbm, stab_hbm, alut_hbm, semb_hbm, acont_hbm,
               nidx_v, aidx_v, srow_v, arow_v, sem_s, sem_a):
    wid = lax.axis_index("s") * NC + lax.axis_index("c")
    base = wid * BPW
    for j in range(NCH):
        off = base + j * CH
        pltpu.sync_copy(node_hbm.at[pl.ds(off, CH)], nidx_v)
        pltpu.sync_copy(attr_hbm.at[pl.ds(off, CH)], aidx_v)
        cs = pltpu.async_copy(stab_hbm.at[nidx_v], srow_v, sem_s)
        ca = pltpu.async_copy(alut_hbm.at[aidx_v], arow_v, sem_a)
        cs.wait()
        ca.wait()
        pltpu.sync_copy(srow_v, semb_hbm.at[pl.ds(off, CH)])
        pltpu.sync_copy(arow_v, acont_hbm.at[pl.ds(off, CH)])


def kernel(node_ids, attrs, struct_table, attr_table, attr_fc_w, attr_fc_b, fusion_w, fusion_b):
    w1 = fusion_w[:D]
    w2 = fusion_w[D:]
    at_pad = jnp.pad(attr_table, ((0, 7), (0, 0)))

    attr_lut = pl.pallas_call(
        _lut_body,
        out_shape=jax.ShapeDtypeStruct((1008, D), jnp.float32),
    )(at_pad, attr_fc_w, attr_fc_b.reshape(1, D), w2, fusion_b.reshape(1, D))

    mesh = plsc.VectorSubcoreMesh(core_axis_name="c", subcore_axis_name="s",
                                  num_cores=NC, num_subcores=NS)
    semb, acont = pl.kernel(
        _sc_gather,
        out_type=[
            jax.ShapeDtypeStruct((B, D), jnp.float32),
            jax.ShapeDtypeStruct((B, D), jnp.float32),
        ],
        mesh=mesh,
        scratch_types=[
            pltpu.VMEM((CH,), jnp.int32),
            pltpu.VMEM((CH,), jnp.int32),
            pltpu.VMEM((CH, D), jnp.float32),
            pltpu.VMEM((CH, D), jnp.float32),
            pltpu.SemaphoreType.DMA,
            pltpu.SemaphoreType.DMA,
        ],
    )(node_ids, attrs, struct_table, attr_lut)

    BLK = 2048
    out = pl.pallas_call(
        _fuse_body,
        grid=(B // BLK,),
        in_specs=[
            pl.BlockSpec((BLK, D), lambda i: (i, 0)),
            pl.BlockSpec((D, D), lambda i: (0, 0)),
            pl.BlockSpec((BLK, D), lambda i: (i, 0)),
        ],
        out_specs=pl.BlockSpec((BLK, D), lambda i: (i, 0)),
        out_shape=jax.ShapeDtypeStruct((B, D), jnp.float32),
    )(semb, w1, acont)
    return out
